# baseline (speedup 1.0000x reference)
import functools
import math

import jax
import jax.numpy as jnp
from jax.experimental import pallas as pl
from jax.experimental.pallas import tpu as pltpu


def _rms(x, eps=1e-6):
    return x * jax.lax.rsqrt(jnp.mean(x * x, axis=-1, keepdims=True) + eps)


def _qkv_body(h_ref, w_ref, b_ref, o_ref):
    x = _rms(h_ref[...]).astype(jnp.bfloat16)
    o = jnp.dot(x, w_ref[...], preferred_element_type=jnp.float32)
    o_ref[...] = (o + b_ref[...]).astype(jnp.bfloat16)


def _attn_body(q_ref, k_ref, v_ref, o_ref, *, qb, hd):
    i = pl.program_id(1)
    s = jax.lax.dot_general(q_ref[...], k_ref[...], (((1,), (1,)), ((), ())),
                            preferred_element_type=jnp.float32
                            ).astype(jnp.bfloat16)
    S = s.shape[1]
    rows = jax.lax.broadcasted_iota(jnp.int32, (qb, S), 0)
    cols = jax.lax.broadcasted_iota(jnp.int32, (qb, S), 1)
    p = jnp.where(i * qb + rows >= cols,
                  jnp.exp(jnp.minimum(s, jnp.bfloat16(60.0))),
                  jnp.bfloat16(0.0))
    ctx = jnp.dot(p, v_ref[...], preferred_element_type=jnp.float32)
    l = ctx[:, hd:hd + 1]
    o_ref[...] = (ctx / l).astype(jnp.bfloat16)


def _post_body(ctx_ref, wo_ref, bo_ref, hid_ref, wr_ref, br_ref,
               h1_ref, x_ref, p_ref):
    ao = jnp.dot(ctx_ref[...], wo_ref[...],
                 preferred_element_type=jnp.float32) + bo_ref[...]
    h1 = hid_ref[...] + ao
    h1_ref[...] = h1
    x = _rms(h1).astype(jnp.bfloat16)
    x_ref[...] = x
    lg = jnp.dot(x, wr_ref[...], preferred_element_type=jnp.float32) + br_ref[...]
    lg = lg - jnp.max(lg, axis=-1, keepdims=True)
    ex = jnp.exp(lg)
    p_ref[...] = ex / jnp.sum(ex, axis=-1, keepdims=True)


def _moe_body(x_ref, p_ref, w1_ref, b1_ref, w2_ref, b2_ref, o_ref, *, n_exp):
    e = pl.program_id(0)
    f = pl.program_id(1)
    w1 = w1_ref[0].astype(jnp.bfloat16)
    h = jnp.dot(x_ref[...], w1, preferred_element_type=jnp.float32)
    h = (h + b1_ref[0]).astype(jnp.bfloat16)
    t = jnp.tanh(h * (jnp.bfloat16(0.7978845608) +
                      jnp.bfloat16(0.0356774081) * h * h))
    g = jnp.bfloat16(0.5) * h * (jnp.bfloat16(1.0) + t)
    probs = p_ref[...]
    sel = (jax.lax.broadcasted_iota(jnp.int32, (1, n_exp), 1) == e)
    pe = jnp.sum(jnp.where(sel, probs, 0.0), axis=-1, keepdims=True)
    g = g * pe.astype(jnp.bfloat16)
    w2 = w2_ref[0].astype(jnp.bfloat16)
    y = jnp.dot(g, w2, preferred_element_type=jnp.float32)

    @pl.when((e == 0) & (f == 0))
    def _():
        o_ref[...] = (jnp.dot(probs, b2_ref[...],
                              preferred_element_type=jnp.float32)
                      + y).astype(jnp.bfloat16)

    @pl.when((e != 0) | (f != 0))
    def _():
        o_ref[...] += y.astype(jnp.bfloat16)


def _forward_impl(h2d, Wqkv, bqkv, Wo, bo, hid, Wr, br, We1, be1, We2, be2,
                  n_heads, hd, interpret=False):
    S, D = h2d.shape
    E, _, F = We1.shape
    hdp = 128
    Dp = n_heads * hdp

    tb1 = min(512, S)
    qkv = pl.pallas_call(
        _qkv_body,
        grid=(S // tb1,),
        in_specs=[
            pl.BlockSpec((tb1, D), lambda t: (t, 0)),
            pl.BlockSpec((D, 3 * Dp), lambda t: (0, 0)),
            pl.BlockSpec((1, 3 * Dp), lambda t: (0, 0)),
        ],
        out_specs=pl.BlockSpec((tb1, 3 * Dp), lambda t: (t, 0)),
        out_shape=jax.ShapeDtypeStruct((S, 3 * Dp), jnp.bfloat16),
        interpret=interpret,
    )(h2d, Wqkv, bqkv)

    qb = min(1024, S)
    ctx = pl.pallas_call(
        functools.partial(_attn_body, qb=qb, hd=hd),
        grid=(n_heads, S // qb),
        in_specs=[
            pl.BlockSpec((qb, hdp), lambda h, i: (i, h)),
            pl.BlockSpec((S, hdp), lambda h, i: (0, n_heads + h)),
            pl.BlockSpec((S, hdp), lambda h, i: (0, 2 * n_heads + h)),
        ],
        out_specs=pl.BlockSpec((qb, hdp), lambda h, i: (i, h)),
        out_shape=jax.ShapeDtypeStruct((S, Dp), jnp.bfloat16),
        interpret=interpret,
    )(qkv, qkv, qkv)

    tb2 = min(1024, S)
    h1, xn, probs = pl.pallas_call(
        _post_body,
        grid=(S // tb2,),
        in_specs=[
            pl.BlockSpec((tb2, Dp), lambda t: (t, 0)),
            pl.BlockSpec((Dp, D), lambda t: (0, 0)),
            pl.BlockSpec((1, D), lambda t: (0, 0)),
            pl.BlockSpec((tb2, D), lambda t: (t, 0)),
            pl.BlockSpec((D, E), lambda t: (0, 0)),
            pl.BlockSpec((1, E), lambda t: (0, 0)),
        ],
        out_specs=[
            pl.BlockSpec((tb2, D), lambda t: (t, 0)),
            pl.BlockSpec((tb2, D), lambda t: (t, 0)),
            pl.BlockSpec((tb2, E), lambda t: (t, 0)),
        ],
        out_shape=[
            jax.ShapeDtypeStruct((S, D), jnp.float32),
            jax.ShapeDtypeStruct((S, D), jnp.bfloat16),
            jax.ShapeDtypeStruct((S, E), jnp.float32),
        ],
        interpret=interpret,
    )(ctx, Wo, bo, hid, Wr, br)

    f2 = min(1024, F)
    moe = pl.pallas_call(
        functools.partial(_moe_body, n_exp=E),
        grid=(E, F // f2),
        in_specs=[
            pl.BlockSpec((S, D), lambda e, f: (0, 0)),
            pl.BlockSpec((S, E), lambda e, f: (0, 0)),
            pl.BlockSpec((1, D, f2), lambda e, f: (e, 0, f)),
            pl.BlockSpec((1, 1, f2), lambda e, f: (e, 0, f)),
            pl.BlockSpec((1, f2, D), lambda e, f: (e, f, 0)),
            pl.BlockSpec((E, D), lambda e, f: (0, 0)),
        ],
        out_specs=pl.BlockSpec((S, D), lambda e, f: (0, 0)),
        out_shape=jax.ShapeDtypeStruct((S, D), jnp.bfloat16),
        interpret=interpret,
    )(xn, probs, We1, be1.reshape(E, 1, F), We2, be2)
    return h1 + moe.astype(jnp.float32)


def _pad_heads(w, n_heads, hd, hdp):
    D = w.shape[0]
    w = w.reshape(D, n_heads, hd)
    w = jnp.pad(w, ((0, 0), (0, 0), (0, hdp - hd)))
    return w.reshape(D, n_heads * hdp)


def kernel(hidden_states, attention_mask, cache, Wq, bq, Wk, bk, Wv, bv,
           Wo, bo, Wr, br, We1, be1, We2, be2):
    del attention_mask, cache
    B, S, D = hidden_states.shape
    H = 16
    hd = D // H
    hdp = 128
    h2d = hidden_states.reshape(B * S, D)
    scale = 1.0 / math.sqrt(hd)
    Wqkv = jnp.concatenate(
        [_pad_heads(Wq * scale, H, hd, hdp),
         _pad_heads(Wk, H, hd, hdp),
         _pad_heads(Wv, H, hd, hdp)],
        axis=1).astype(jnp.bfloat16)
    bqkv = jnp.concatenate(
        [_pad_heads((bq * scale).reshape(1, D), H, hd, hdp),
         _pad_heads(bk.reshape(1, D), H, hd, hdp),
         _pad_heads(bv.reshape(1, D), H, hd, hdp)],
        axis=1)
    lane = jnp.arange(3 * H * hdp)
    is_v_ones = (lane >= 2 * H * hdp) & (lane % hdp == hd)
    bqkv = jnp.where(is_v_ones[None, :], 1.0, bqkv)
    Wo_p = jnp.pad(Wo.reshape(H, hd, D), ((0, 0), (0, hdp - hd), (0, 0)))
    Wo_p = Wo_p.reshape(H * hdp, D).astype(jnp.bfloat16)
    out = _forward_impl(
        h2d, Wqkv, bqkv, Wo_p, bo.reshape(1, D), h2d,
        Wr.astype(jnp.bfloat16), br.reshape(1, -1),
        We1, be1, We2, be2,
        n_heads=H, hd=hd)
    return out.reshape(B, S, D)


# speedup vs baseline: 1.0137x; 1.0137x over previous
import functools
import math

import jax
import jax.numpy as jnp
from jax.experimental import pallas as pl
from jax.experimental.pallas import tpu as pltpu


def _rms(x, eps=1e-6):
    return x * jax.lax.rsqrt(jnp.mean(x * x, axis=-1, keepdims=True) + eps)


def _qkv_body(h_ref, w_ref, b_ref, o_ref):
    x = _rms(h_ref[...]).astype(jnp.bfloat16)
    o = jnp.dot(x, w_ref[...], preferred_element_type=jnp.float32)
    o_ref[...] = (o + b_ref[...]).astype(jnp.bfloat16)


def _attn_body(q_ref, k_ref, v_ref, o_ref, *, qb, hd, chunks=2):
    i = pl.program_id(1)
    k = k_ref[...]
    v = v_ref[...]
    S = k.shape[0]
    cb = qb // chunks
    rows = jax.lax.broadcasted_iota(jnp.int32, (cb, S), 0)
    cols = jax.lax.broadcasted_iota(jnp.int32, (cb, S), 1)
    for c in range(chunks):
        q = q_ref[c * cb:(c + 1) * cb, :]
        s = jax.lax.dot_general(q, k, (((1,), (1,)), ((), ())),
                                preferred_element_type=jnp.float32
                                ).astype(jnp.bfloat16)
        p = jnp.where(i * qb + c * cb + rows >= cols,
                      jnp.exp(jnp.minimum(s, jnp.bfloat16(60.0))),
                      jnp.bfloat16(0.0))
        ctx = jnp.dot(p, v, preferred_element_type=jnp.float32)
        l = ctx[:, hd:hd + 1]
        o_ref[c * cb:(c + 1) * cb, :] = (ctx / l).astype(jnp.bfloat16)


def _post_body(ctx_ref, wo_ref, bo_ref, hid_ref, wr_ref, br_ref,
               h1_ref, x_ref, p_ref):
    ao = jnp.dot(ctx_ref[...], wo_ref[...],
                 preferred_element_type=jnp.float32) + bo_ref[...]
    h1 = hid_ref[...] + ao
    h1_ref[...] = h1
    x = _rms(h1).astype(jnp.bfloat16)
    x_ref[...] = x
    lg = jnp.dot(x, wr_ref[...], preferred_element_type=jnp.float32) + br_ref[...]
    lg = lg - jnp.max(lg, axis=-1, keepdims=True)
    ex = jnp.exp(lg)
    p_ref[...] = ex / jnp.sum(ex, axis=-1, keepdims=True)


def _moe_body(x_ref, p_ref, w1_ref, b1_ref, w2_ref, b2_ref, o_ref,
              *, n_exp, chunks=2):
    e = pl.program_id(0)
    f = pl.program_id(1)
    w1 = w1_ref[0].astype(jnp.bfloat16)
    w2 = w2_ref[0].astype(jnp.bfloat16)
    b1 = b1_ref[0]
    sel = (jax.lax.broadcasted_iota(jnp.int32, (1, n_exp), 1) == e)
    S = x_ref.shape[0]
    cb = S // chunks
    for c in range(chunks):
        r = slice(c * cb, (c + 1) * cb)
        h = jnp.dot(x_ref[r, :], w1, preferred_element_type=jnp.float32)
        h = (h + b1).astype(jnp.bfloat16)
        t = jnp.tanh(h * (jnp.bfloat16(0.7978845608) +
                          jnp.bfloat16(0.0356774081) * h * h))
        g = jnp.bfloat16(0.5) * h * (jnp.bfloat16(1.0) + t)
        probs = p_ref[r, :]
        pe = jnp.sum(jnp.where(sel, probs, 0.0), axis=-1, keepdims=True)
        g = g * pe.astype(jnp.bfloat16)
        y = jnp.dot(g, w2, preferred_element_type=jnp.float32)

        @pl.when((e == 0) & (f == 0))
        def _():
            o_ref[r, :] = (jnp.dot(probs, b2_ref[...],
                                   preferred_element_type=jnp.float32)
                           + y).astype(jnp.bfloat16)

        @pl.when((e != 0) | (f != 0))
        def _():
            o_ref[r, :] += y.astype(jnp.bfloat16)


def _forward_impl(h2d, Wqkv, bqkv, Wo, bo, hid, Wr, br, We1, be1, We2, be2,
                  n_heads, hd, interpret=False):
    S, D = h2d.shape
    E, _, F = We1.shape
    hdp = 128
    Dp = n_heads * hdp

    tb1 = min(512, S)
    qkv = pl.pallas_call(
        _qkv_body,
        grid=(S // tb1,),
        in_specs=[
            pl.BlockSpec((tb1, D), lambda t: (t, 0)),
            pl.BlockSpec((D, 3 * Dp), lambda t: (0, 0)),
            pl.BlockSpec((1, 3 * Dp), lambda t: (0, 0)),
        ],
        out_specs=pl.BlockSpec((tb1, 3 * Dp), lambda t: (t, 0)),
        out_shape=jax.ShapeDtypeStruct((S, 3 * Dp), jnp.bfloat16),
        interpret=interpret,
    )(h2d, Wqkv, bqkv)

    qb = min(1024, S)
    ctx = pl.pallas_call(
        functools.partial(_attn_body, qb=qb, hd=hd),
        grid=(n_heads, S // qb),
        in_specs=[
            pl.BlockSpec((qb, hdp), lambda h, i: (i, h)),
            pl.BlockSpec((S, hdp), lambda h, i: (0, n_heads + h)),
            pl.BlockSpec((S, hdp), lambda h, i: (0, 2 * n_heads + h)),
        ],
        out_specs=pl.BlockSpec((qb, hdp), lambda h, i: (i, h)),
        out_shape=jax.ShapeDtypeStruct((S, Dp), jnp.bfloat16),
        interpret=interpret,
    )(qkv, qkv, qkv)

    tb2 = min(1024, S)
    h1, xn, probs = pl.pallas_call(
        _post_body,
        grid=(S // tb2,),
        in_specs=[
            pl.BlockSpec((tb2, Dp), lambda t: (t, 0)),
            pl.BlockSpec((Dp, D), lambda t: (0, 0)),
            pl.BlockSpec((1, D), lambda t: (0, 0)),
            pl.BlockSpec((tb2, D), lambda t: (t, 0)),
            pl.BlockSpec((D, E), lambda t: (0, 0)),
            pl.BlockSpec((1, E), lambda t: (0, 0)),
        ],
        out_specs=[
            pl.BlockSpec((tb2, D), lambda t: (t, 0)),
            pl.BlockSpec((tb2, D), lambda t: (t, 0)),
            pl.BlockSpec((tb2, E), lambda t: (t, 0)),
        ],
        out_shape=[
            jax.ShapeDtypeStruct((S, D), jnp.float32),
            jax.ShapeDtypeStruct((S, D), jnp.bfloat16),
            jax.ShapeDtypeStruct((S, E), jnp.float32),
        ],
        interpret=interpret,
    )(ctx, Wo, bo, hid, Wr, br)

    f2 = min(1024, F)
    moe = pl.pallas_call(
        functools.partial(_moe_body, n_exp=E),
        grid=(E, F // f2),
        in_specs=[
            pl.BlockSpec((S, D), lambda e, f: (0, 0)),
            pl.BlockSpec((S, E), lambda e, f: (0, 0)),
            pl.BlockSpec((1, D, f2), lambda e, f: (e, 0, f)),
            pl.BlockSpec((1, 1, f2), lambda e, f: (e, 0, f)),
            pl.BlockSpec((1, f2, D), lambda e, f: (e, f, 0)),
            pl.BlockSpec((E, D), lambda e, f: (0, 0)),
        ],
        out_specs=pl.BlockSpec((S, D), lambda e, f: (0, 0)),
        out_shape=jax.ShapeDtypeStruct((S, D), jnp.bfloat16),
        interpret=interpret,
    )(xn, probs, We1, be1.reshape(E, 1, F), We2, be2)
    return h1 + moe.astype(jnp.float32)


def _pad_heads(w, n_heads, hd, hdp):
    D = w.shape[0]
    w = w.reshape(D, n_heads, hd)
    w = jnp.pad(w, ((0, 0), (0, 0), (0, hdp - hd)))
    return w.reshape(D, n_heads * hdp)


def kernel(hidden_states, attention_mask, cache, Wq, bq, Wk, bk, Wv, bv,
           Wo, bo, Wr, br, We1, be1, We2, be2):
    del attention_mask, cache
    B, S, D = hidden_states.shape
    H = 16
    hd = D // H
    hdp = 128
    h2d = hidden_states.reshape(B * S, D)
    scale = 1.0 / math.sqrt(hd)
    Wqkv = jnp.concatenate(
        [_pad_heads(Wq * scale, H, hd, hdp),
         _pad_heads(Wk, H, hd, hdp),
         _pad_heads(Wv, H, hd, hdp)],
        axis=1).astype(jnp.bfloat16)
    bqkv = jnp.concatenate(
        [_pad_heads((bq * scale).reshape(1, D), H, hd, hdp),
         _pad_heads(bk.reshape(1, D), H, hd, hdp),
         _pad_heads(bv.reshape(1, D), H, hd, hdp)],
        axis=1)
    lane = jnp.arange(3 * H * hdp)
    is_v_ones = (lane >= 2 * H * hdp) & (lane % hdp == hd)
    bqkv = jnp.where(is_v_ones[None, :], 1.0, bqkv)
    Wo_p = jnp.pad(Wo.reshape(H, hd, D), ((0, 0), (0, hdp - hd), (0, 0)))
    Wo_p = Wo_p.reshape(H * hdp, D).astype(jnp.bfloat16)
    out = _forward_impl(
        h2d, Wqkv, bqkv, Wo_p, bo.reshape(1, D), h2d,
        Wr.astype(jnp.bfloat16), br.reshape(1, -1),
        We1, be1, We2, be2,
        n_heads=H, hd=hd)
    return out.reshape(B, S, D)
